# Initial kernel scaffold; baseline (speedup 1.0000x reference)
import jax
import jax.numpy as jnp
from jax.experimental import pallas as pl


def kernel(x, L_sym, W1, b1, W2, b2, theta):
    raise NotImplementedError("write your pallas kernel here")


# speedup vs baseline: 1.0310x; 1.0310x over previous
import functools

import jax
import jax.numpy as jnp
import numpy as np
from jax.experimental import pallas as pl
from jax.experimental.pallas import tpu as pltpu

K = 8
KQ = 256


def _binom() -> np.ndarray:
    from math import comb
    return np.array([float(comb(K, k)) for k in range(K + 1)], dtype=np.float32)


def _dot_l(l_ref, t_ref, width):
    n = l_ref.shape[1]

    def step(i, acc):
        return acc + jnp.dot(
            l_ref[:, pl.ds(i * KQ, KQ)], t_ref[pl.ds(i * KQ, KQ), :width],
            preferred_element_type=jnp.float32)

    init = jnp.dot(l_ref[:, 0:KQ], t_ref[0:KQ, :width],
                   preferred_element_type=jnp.float32)
    return jax.lax.fori_loop(1, n // KQ, step, init)


def _bern_body(coef_ref, x_ref, w1_ref, b1_ref, w2_ref, b2_ref, l_ref, z_ref,
               pw_ref, t_ref):
    a = jnp.dot(x_ref[...], w1_ref[...], preferred_element_type=jnp.float32)
    a = jnp.maximum(a + b1_ref[...], 0.0)
    h = jnp.dot(a, w2_ref[...], preferred_element_type=jnp.float32) + b2_ref[...]
    c = h.shape[1]
    pw_ref[0] = h
    for k in range(1, K + 1):
        pw_ref[k] = _dot_l(l_ref, pw_ref.at[k - 1], c)
    z = jnp.zeros_like(h)
    for k in range(K + 1):
        t_ref[...] = pw_ref[k]
        for _ in range(K - k):
            p = _dot_l(l_ref, t_ref, c)
            t_ref[...] = 2.0 * t_ref[...] - p
        z = z + coef_ref[k] * t_ref[...]
    z_ref[...] = z


@functools.partial(jax.jit, static_argnames=("interpret",))
def kernel(x, L_sym, W1, b1, W2, b2, theta, interpret=False):
    n, c_dim = x.shape[0], W2.shape[1]
    coef = jnp.maximum(theta, 0.0) * jnp.asarray(_binom()) / (2.0 ** K)
    smem = pl.BlockSpec(memory_space=pltpu.SMEM)
    return pl.pallas_call(
        _bern_body,
        out_shape=jax.ShapeDtypeStruct((n, c_dim), jnp.float32),
        in_specs=[smem] + [pl.BlockSpec()] * 6,
        out_specs=pl.BlockSpec(),
        scratch_shapes=[
            pltpu.VMEM((K + 1, n, c_dim), jnp.float32),
            pltpu.VMEM((n, c_dim), jnp.float32),
        ],
        compiler_params=pltpu.CompilerParams(
            vmem_limit_bytes=64 * 1024 * 1024,
        ),
        interpret=interpret,
    )(coef, x, W1, b1.reshape(1, -1), W2, b2.reshape(1, -1), L_sym)


# speedup vs baseline: 2.8613x; 2.7754x over previous
import functools

import jax
import jax.numpy as jnp
import numpy as np
from jax.experimental import pallas as pl
from jax.experimental.pallas import tpu as pltpu

K = 8
KQ = 256
CB = 64


def _binom() -> np.ndarray:
    from math import comb
    return np.array([float(comb(K, k)) for k in range(K + 1)], dtype=np.float32)


def _dot_l(l_ref, u_ref, width):
    n = l_ref.shape[1]

    def step(i, acc):
        return acc + jnp.dot(
            l_ref[:, pl.ds(i * KQ, KQ)], u_ref[pl.ds(i * KQ, KQ), :width],
            preferred_element_type=jnp.float32)

    init = jnp.dot(l_ref[:, 0:KQ], u_ref[0:KQ, :width],
                   preferred_element_type=jnp.float32)
    return jax.lax.fori_loop(1, n // KQ, step, init)


def _bern_body(coef_ref, x_ref, w1_ref, b1_ref, w2_ref, b2_ref, l_ref, z_ref,
               u_ref):
    a = jnp.dot(x_ref[...], w1_ref[...], preferred_element_type=jnp.float32)
    a = jnp.maximum(a + b1_ref[...], 0.0)
    h = jnp.dot(a, w2_ref[...], preferred_element_type=jnp.float32) + b2_ref[...]
    u_ref[:, 0:CB] = h
    last_term = None
    last_pow = None
    for s in range(K):
        w = CB * (s + 1)
        r = _dot_l(l_ref, u_ref, w)
        r0 = r[:, 0:CB]
        new_term = 2.0 * u_ref[:, 0:CB] - r0
        for j in range(1, s + 1):
            u_ref[:, CB * j:CB * (j + 1)] = (
                2.0 * u_ref[:, CB * j:CB * (j + 1)] - r[:, CB * j:CB * (j + 1)])
        if s < K - 1:
            u_ref[:, CB * (s + 1):CB * (s + 2)] = new_term
            u_ref[:, 0:CB] = r0
        else:
            last_term = new_term
            last_pow = r0
    z = jnp.zeros_like(h)
    for k in range(K - 1):
        z = z + coef_ref[k] * u_ref[:, CB * (k + 1):CB * (k + 2)]
    z = z + coef_ref[K - 1] * last_term
    z = z + coef_ref[K] * last_pow
    z_ref[...] = z


@functools.partial(jax.jit, static_argnames=("interpret",))
def kernel(x, L_sym, W1, b1, W2, b2, theta, interpret=False):
    n, c_dim = x.shape[0], W2.shape[1]
    coef = jnp.maximum(theta, 0.0) * jnp.asarray(_binom()) / (2.0 ** K)
    smem = pl.BlockSpec(memory_space=pltpu.SMEM)
    return pl.pallas_call(
        _bern_body,
        out_shape=jax.ShapeDtypeStruct((n, c_dim), jnp.float32),
        in_specs=[smem] + [pl.BlockSpec()] * 6,
        out_specs=pl.BlockSpec(),
        scratch_shapes=[
            pltpu.VMEM((n, CB * K), jnp.float32),
        ],
        compiler_params=pltpu.CompilerParams(
            vmem_limit_bytes=64 * 1024 * 1024,
        ),
        interpret=interpret,
    )(coef, x, W1, b1.reshape(1, -1), W2, b2.reshape(1, -1), L_sym)
